# Initial kernel scaffold; baseline (speedup 1.0000x reference)
import jax
import jax.numpy as jnp
from jax.experimental import pallas as pl


def kernel(features, seq_lengths, W1, att_src1, att_dst1, b1, W2, att_src2, att_dst2, b2, W_ih, W_hh, b_ih, b_hh, W_cls, b_cls):
    raise NotImplementedError("write your pallas kernel here")


# speedup vs baseline: 1.6170x; 1.6170x over previous
import functools

import jax
import jax.numpy as jnp
from jax.experimental import pallas as pl
from jax.experimental.pallas import tpu as pltpu

A = 23
B = 8
T = 128
F_IN = 16
GNN_HIDDEN = 32
HEADS = 2
LSTM_HIDDEN = 64

NF = B * T
TILE = 128
NB = NF // TILE


def _attention(Hh, att_s, att_d):
    a_src = jnp.sum(Hh * att_s[None, None, :], axis=-1)
    a_dst = jnp.sum(Hh * att_d[None, None, :], axis=-1)
    e = a_src[:, None, :] + a_dst[None, :, :]
    e = jnp.where(e >= 0, e, 0.2 * e)
    m = jnp.max(e, axis=0)
    p = jnp.exp(e - m[None, :, :])
    s = jnp.sum(p, axis=0)
    alpha = p / (s[None, :, :] + 1e-16)
    acc = jnp.zeros(Hh.shape, jnp.float32)
    for i in range(A):
        acc = acc + alpha[i][:, :, None] * Hh[i][None, :, :]
    return acc


def _gat_kernel(x_ref, w1_ref, as1_ref, ad1_ref, b1_ref,
                w2_ref, as2_ref, ad2_ref, b2_ref, out_ref):
    x = x_ref[:]
    H = jnp.dot(x.reshape(A * TILE, F_IN), w1_ref[:],
                preferred_element_type=jnp.float32)
    H = H.reshape(A, TILE, HEADS * GNN_HIDDEN)
    outs = []
    for h in range(HEADS):
        Hh = H[:, :, h * GNN_HIDDEN:(h + 1) * GNN_HIDDEN]
        outs.append(_attention(Hh, as1_ref[h, :], ad1_ref[h, :]))
    x2 = jnp.concatenate(outs, axis=-1) + b1_ref[0, :][None, None, :]
    x2 = jnp.maximum(x2, 0.0)

    H2 = jnp.dot(x2.reshape(A * TILE, HEADS * GNN_HIDDEN), w2_ref[:],
                 preferred_element_type=jnp.float32)
    H2 = H2.reshape(A, TILE, GNN_HIDDEN)
    o2 = _attention(H2, as2_ref[0, :], ad2_ref[0, :])
    o2 = jnp.maximum(o2 + b2_ref[0, :][None, None, :], 0.0)
    out_ref[:] = jnp.sum(o2, axis=0) * (1.0 / A)


def _lstm_kernel(xs_ref, seq_ref, wih_ref, whh_ref, bias_ref,
                 wcls_ref, bcls_ref, out_ref, xg_ref):
    xg_ref[:] = (jnp.dot(xs_ref[:], wih_ref[:],
                         preferred_element_type=jnp.float32)
                 + bias_ref[0, :][None, :]).reshape(T, B, 4 * LSTM_HIDDEN)
    seq = seq_ref[:]

    def step(t, carry):
        h, c = carry
        gates = xg_ref[t] + jnp.dot(h, whh_ref[:],
                                    preferred_element_type=jnp.float32)
        i = jax.nn.sigmoid(gates[:, 0 * LSTM_HIDDEN:1 * LSTM_HIDDEN])
        f = jax.nn.sigmoid(gates[:, 1 * LSTM_HIDDEN:2 * LSTM_HIDDEN])
        g = jnp.tanh(gates[:, 2 * LSTM_HIDDEN:3 * LSTM_HIDDEN])
        o = jax.nn.sigmoid(gates[:, 3 * LSTM_HIDDEN:4 * LSTM_HIDDEN])
        c_new = f * c + i * g
        h_new = o * jnp.tanh(c_new)
        valid = t < seq
        return jnp.where(valid, h_new, h), jnp.where(valid, c_new, c)

    zero = jnp.zeros((B, LSTM_HIDDEN), jnp.float32)
    h_last, _ = jax.lax.fori_loop(0, T, step, (zero, zero))
    out_ref[:] = (jnp.sum(h_last * wcls_ref[:], axis=-1, keepdims=True)
                  + bcls_ref[0, :][None, :])


@functools.partial(jax.jit, static_argnames=())
def kernel(features, seq_lengths, W1, att_src1, att_dst1, b1,
           W2, att_src2, att_dst2, b2, W_ih, W_hh, b_ih, b_hh, W_cls, b_cls):
    xT = features.reshape(NF, A, F_IN).transpose(1, 0, 2)

    pooled = pl.pallas_call(
        _gat_kernel,
        grid=(NB,),
        in_specs=[
            pl.BlockSpec((A, TILE, F_IN), lambda i: (0, i, 0)),
            pl.BlockSpec((F_IN, HEADS * GNN_HIDDEN), lambda i: (0, 0)),
            pl.BlockSpec((HEADS, GNN_HIDDEN), lambda i: (0, 0)),
            pl.BlockSpec((HEADS, GNN_HIDDEN), lambda i: (0, 0)),
            pl.BlockSpec((1, HEADS * GNN_HIDDEN), lambda i: (0, 0)),
            pl.BlockSpec((HEADS * GNN_HIDDEN, GNN_HIDDEN), lambda i: (0, 0)),
            pl.BlockSpec((1, GNN_HIDDEN), lambda i: (0, 0)),
            pl.BlockSpec((1, GNN_HIDDEN), lambda i: (0, 0)),
            pl.BlockSpec((1, GNN_HIDDEN), lambda i: (0, 0)),
        ],
        out_specs=pl.BlockSpec((TILE, GNN_HIDDEN), lambda i: (i, 0)),
        out_shape=jax.ShapeDtypeStruct((NF, GNN_HIDDEN), jnp.float32),
    )(xT, W1, att_src1, att_dst1, b1.reshape(1, -1),
      W2, att_src2, att_dst2, b2.reshape(1, -1))

    xs = pooled.reshape(B, T, GNN_HIDDEN).transpose(1, 0, 2)
    xs = xs.reshape(T * B, GNN_HIDDEN)

    logits = pl.pallas_call(
        _lstm_kernel,
        grid=(1,),
        in_specs=[
            pl.BlockSpec((T * B, GNN_HIDDEN), lambda i: (0, 0)),
            pl.BlockSpec((B, 1), lambda i: (0, 0)),
            pl.BlockSpec((GNN_HIDDEN, 4 * LSTM_HIDDEN), lambda i: (0, 0)),
            pl.BlockSpec((LSTM_HIDDEN, 4 * LSTM_HIDDEN), lambda i: (0, 0)),
            pl.BlockSpec((1, 4 * LSTM_HIDDEN), lambda i: (0, 0)),
            pl.BlockSpec((1, LSTM_HIDDEN), lambda i: (0, 0)),
            pl.BlockSpec((1, 1), lambda i: (0, 0)),
        ],
        out_specs=pl.BlockSpec((B, 1), lambda i: (0, 0)),
        out_shape=jax.ShapeDtypeStruct((B, 1), jnp.float32),
        scratch_shapes=[pltpu.VMEM((T, B, 4 * LSTM_HIDDEN), jnp.float32)],
    )(xs, seq_lengths.reshape(B, 1), W_ih.T, W_hh.T,
      (b_ih + b_hh).reshape(1, -1), W_cls, b_cls.reshape(1, 1))

    return logits.reshape(B)
